# baseline (speedup 1.0000x reference)
import functools
import math

import jax
import jax.numpy as jnp
import numpy as np
from jax import lax
from jax.experimental import pallas as pl
from jax.experimental.pallas import tpu as pltpu
from jax.experimental.pallas import tpu_sc as plsc

_MAX_LEN = 1500
_NC = 2
_NS = 16


def _pe_np(max_len: int, d: int) -> np.ndarray:
    position = np.arange(0, max_len, dtype=np.float32)[:, None]
    div_term = np.exp(
        np.arange(0, d, 2, dtype=np.float32) * -(math.log(10000.0) / d)
    )
    pe = np.zeros((max_len, d), dtype=np.float32)
    pe[:, 0::2] = np.sin(position * div_term)
    pe[:, 1::2] = np.cos(position * div_term)
    return pe


def _prep_body(idx_ref, tab_ref, pe_ref, fused_ref, loc_ref):
    v = tab_ref.shape[0]
    fused_ref[...] = pe_ref[...][:, None, :] + tab_ref[...][None, :, :]
    idx = idx_ref[...]
    pos = lax.broadcasted_iota(jnp.int32, idx.shape, 1)
    loc_ref[...] = idx + pos * v


def _prep(indices, table, pe):
    b, l = indices.shape
    v, d = table.shape
    return pl.pallas_call(
        _prep_body,
        out_shape=(
            jax.ShapeDtypeStruct((l, v, d), jnp.float32),
            jax.ShapeDtypeStruct((b, l), jnp.int32),
        ),
    )(indices, table, pe)


def _sc_gather(fused, loc, l):
    total = loc.shape[0] // l
    d = fused.shape[1]
    rows_t = fused.shape[0]
    per_core = total // _NC
    per_w = per_core // _NS
    nb = 2
    n4 = per_w // nb
    mesh = plsc.VectorSubcoreMesh(core_axis_name="c", subcore_axis_name="s")

    @functools.partial(
        pl.kernel,
        mesh=mesh,
        out_type=jax.ShapeDtypeStruct((total, l, d), jnp.float32),
        scratch_types=[
            pltpu.VMEM((per_w * l,), jnp.int32),
            pltpu.VMEM((nb, l, d), jnp.float32),
            pltpu.VMEM_SHARED((rows_t, d), jnp.float32),
        ]
        + [pltpu.SemaphoreType.DMA] * (2 * nb),
    )
    def k(fused_hbm, loc_hbm, out_hbm, idx_v, rows_v, fused_sp, *sems):
        gsems, wsems = sems[:nb], sems[nb:]
        cid = lax.axis_index("c")
        sid = lax.axis_index("s")
        b0 = cid * per_core + sid * per_w

        @pl.when(sid == 0)
        def _():
            pltpu.sync_copy(fused_hbm, fused_sp)

        pltpu.sync_copy(loc_hbm.at[pl.ds(b0 * l, per_w * l)], idx_v)
        plsc.subcore_barrier()

        def gather(i, b):
            return pltpu.make_async_copy(
                fused_sp.at[idx_v.at[pl.ds(i * l, l)]],
                rows_v.at[b],
                gsems[b],
            )

        def write(i, b):
            return pltpu.make_async_copy(
                rows_v.at[b],
                out_hbm.at[b0 + i],
                wsems[b],
            )

        for b in range(nb - 1):
            gather(b, b).start()

        def body(i4, carry):
            for b in range(nb):
                i = i4 * nb + b
                gather(i, b).wait()
                write(i, b).start()
                bm1 = (b - 1) % nb
                if b == 0:

                    @pl.when(i4 > 0)
                    def _():
                        write(i - 1, bm1).wait()

                    gather(i + nb - 1, bm1).start()
                else:
                    write(i - 1, bm1).wait()

                    @pl.when(i4 < n4 - 1)
                    def _():
                        gather(i + nb - 1, bm1).start()

            return carry

        lax.fori_loop(0, n4, body, 0)
        write(per_w - 1, (per_w - 1) % nb).wait()

    return k(fused, loc)


def kernel(indices, table):
    b, l = indices.shape
    v, d = table.shape
    pe = jnp.asarray(_pe_np(_MAX_LEN, d)[:l])
    fused, loc = _prep(indices, table, pe)
    return _sc_gather(fused.reshape(l * v, d), loc.reshape(b * l), l)
